# baseline (speedup 1.0000x reference)
import dataclasses
import functools

import jax
import jax.numpy as jnp
from jax import lax
from jax.experimental import pallas as pl
from jax.experimental.pallas import tpu as pltpu
from jax.experimental.pallas import tpu_sc as plsc

VOCAB = 50368
HIDDEN = 768
EPS = 1e-05
BATCH = 4
SEQ = 8192

NUM_TOKENS = BATCH * SEQ
NC = 2
NS = 16
NW = NC * NS
B_PER_W = NUM_TOKENS // NW
CHUNK = 64
N_CHUNKS = B_PER_W // CHUNK
NBUF = 2
LANES = 16
NVEC = HIDDEN // LANES


RI = 8


def _layernorm_chunk(buf, wv):
    @pl.loop(0, CHUNK, step=RI)
    def _(r0):
        acc_s = [jnp.zeros((LANES,), jnp.float32) for _ in range(RI)]
        acc_q = [jnp.zeros((LANES,), jnp.float32) for _ in range(RI)]
        for v in range(NVEC):
            sl = pl.ds(v * LANES, LANES)
            for i in range(RI):
                x = buf[r0 + i, sl]
                acc_s[i] = acc_s[i] + x
                acc_q[i] = acc_q[i] + x * x
        s = [jnp.sum(a) for a in acc_s]
        q = [jnp.sum(a) for a in acc_q]
        mean = [si * (1.0 / HIDDEN) for si in s]
        var = [qi * (1.0 / HIDDEN) - mi * mi + EPS for qi, mi in zip(q, mean)]
        y = [lax.bitcast_convert_type(
                jnp.int32(0x5F3759DF) - lax.shift_right_arithmetic(
                    lax.bitcast_convert_type(vi, jnp.int32), 1),
                jnp.float32) for vi in var]
        half_var = [0.5 * vi for vi in var]
        for _ in range(3):
            y = [yi * (1.5 - hi * yi * yi) for yi, hi in zip(y, half_var)]
        for v in range(NVEC):
            sl = pl.ds(v * LANES, LANES)
            wvv = wv[sl]
            for i in range(RI):
                x = buf[r0 + i, sl]
                buf[r0 + i, sl] = (x - mean[i]) * y[i] * wvv


def _sc_gather_layernorm(table, idx_flat, w):
    mesh = plsc.VectorSubcoreMesh(core_axis_name="c", subcore_axis_name="s")
    cp = pltpu.CompilerParams()
    if "needs_layout_passes" in pltpu.CompilerParams.__dataclass_fields__:
        cp = dataclasses.replace(cp, needs_layout_passes=False)

    scratch = [
        pltpu.VMEM((B_PER_W,), jnp.int32),
        pltpu.VMEM((HIDDEN,), jnp.float32),
    ]
    scratch += [pltpu.VMEM((CHUNK, HIDDEN), jnp.float32) for _ in range(NBUF)]
    scratch += [pltpu.SemaphoreType.DMA for _ in range(2 * NBUF)]

    @functools.partial(
        pl.kernel,
        out_type=jax.ShapeDtypeStruct((NUM_TOKENS, HIDDEN), jnp.float32),
        mesh=mesh,
        compiler_params=cp,
        scratch_types=scratch,
    )
    def fused_kernel(table_hbm, idx_hbm, w_hbm, out_hbm, idx_v, wv, *bufsem):
        bufs = bufsem[:NBUF]
        gsems = bufsem[NBUF:2 * NBUF]
        wsems = bufsem[2 * NBUF:]
        wid = lax.axis_index("s") * NC + lax.axis_index("c")
        base = wid * B_PER_W
        pltpu.sync_copy(idx_hbm.at[pl.ds(base, B_PER_W)], idx_v)
        pltpu.sync_copy(w_hbm, wv)

        def start_gather(k, j):
            pltpu.async_copy(
                table_hbm.at[idx_v.at[pl.ds(k * CHUNK, CHUNK)]],
                bufs[j], gsems[j])

        def wait_gather(k, j):
            pltpu.make_async_copy(
                table_hbm.at[idx_v.at[pl.ds(k * CHUNK, CHUNK)]],
                bufs[j], gsems[j]).wait()

        def start_wb(k, j):
            pltpu.async_copy(
                bufs[j], out_hbm.at[pl.ds(base + k * CHUNK, CHUNK)], wsems[j])

        def wait_wb(k, j):
            pltpu.make_async_copy(
                bufs[j], out_hbm.at[pl.ds(base + k * CHUNK, CHUNK)],
                wsems[j]).wait()

        for j in range(NBUF - 1):
            start_gather(j, j)

        @pl.loop(0, N_CHUNKS, step=NBUF)
        def _(c):
            for j in range(NBUF):
                k = c + j
                wait_gather(k, j)
                jn = (j + NBUF - 1) % NBUF

                @pl.when(jnp.logical_and(k > 0, k + NBUF - 1 < N_CHUNKS))
                def _(k=k, jn=jn):
                    wait_wb(k - 1, jn)

                @pl.when(k + NBUF - 1 < N_CHUNKS)
                def _(k=k, jn=jn):
                    start_gather(k + NBUF - 1, jn)

                _layernorm_chunk(bufs[j], wv)
                start_wb(k, j)

        for j in range(NBUF):
            wait_wb(N_CHUNKS - NBUF + j, j)

    return fused_kernel(table, idx_flat, w)


def kernel(input_ids, tok_embeddings, norm_weight):
    idx_flat = input_ids.reshape(NUM_TOKENS)
    normed = _sc_gather_layernorm(tok_embeddings, idx_flat, norm_weight)
    return normed.reshape(BATCH, SEQ, HIDDEN)


# speedup vs baseline: 1.1373x; 1.1373x over previous
import functools

import jax
import jax.numpy as jnp
from jax import lax
from jax.experimental import pallas as pl
from jax.experimental.pallas import tpu as pltpu
from jax.experimental.pallas import tpu_sc as plsc

VOCAB = 50368
HIDDEN = 768
EPS = 1e-05
BATCH = 4
SEQ = 8192

NUM_TOKENS = BATCH * SEQ
NC = 2
NS = 16
NW = NC * NS
CHUNK = 64

GROUP_SIZES = (2048, 6144, 8192, 8192, 6144, 2048)
assert sum(GROUP_SIZES) == NUM_TOKENS
assert all(g % (CHUNK * NW) == 0 for g in GROUP_SIZES)


def _sc_gather_group(table, idx_group, group_tokens):
    mesh = plsc.VectorSubcoreMesh(core_axis_name="c", subcore_axis_name="s")
    b_per_w = group_tokens // NW
    n_chunks = b_per_w // CHUNK

    @functools.partial(
        pl.kernel,
        out_type=jax.ShapeDtypeStruct((group_tokens, HIDDEN), jnp.float32),
        mesh=mesh,
        scratch_types=[
            pltpu.VMEM((b_per_w,), jnp.int32),
            pltpu.VMEM((CHUNK, HIDDEN), jnp.float32),
            pltpu.VMEM((CHUNK, HIDDEN), jnp.float32),
            pltpu.SemaphoreType.DMA,
            pltpu.SemaphoreType.DMA,
        ],
    )
    def gather_kernel(table_hbm, idx_hbm, out_hbm, idx_v, rows_a, rows_b, sem_a, sem_b):
        wid = lax.axis_index("s") * NC + lax.axis_index("c")
        base = wid * b_per_w
        pltpu.sync_copy(idx_hbm.at[pl.ds(base, b_per_w)], idx_v)

        bufs = (rows_a, rows_b)
        sems = (sem_a, sem_b)

        def start(c):
            pltpu.async_copy(
                table_hbm.at[idx_v.at[pl.ds(c * CHUNK, CHUNK)]],
                bufs[c % 2], sems[c % 2])

        start(0)
        for c in range(n_chunks):
            pltpu.make_async_copy(
                table_hbm.at[idx_v.at[pl.ds(c * CHUNK, CHUNK)]],
                bufs[c % 2], sems[c % 2]).wait()
            if c + 1 < n_chunks:
                start(c + 1)
            pltpu.sync_copy(bufs[c % 2],
                            out_hbm.at[pl.ds(base + c * CHUNK, CHUNK)])

    return gather_kernel(table, idx_group)


_LN_BLOCK = 1024


def _ln_body_first(x_ref, w_ref, o_ref):
    x = x_ref[...]
    mean = jnp.mean(x, axis=1, keepdims=True)
    xc = x - mean
    var = jnp.mean(xc * xc, axis=1, keepdims=True)
    o_ref[...] = xc * lax.rsqrt(var + EPS) * w_ref[...]


def _ln_body_chain(x_ref, w_ref, buf_ref, o_ref):
    del buf_ref
    _ln_body_first(x_ref, w_ref, o_ref)


def _tc_layernorm_group(first, row_offset, x_group, w2d, buf):
    n_blocks = x_group.shape[0] // _LN_BLOCK
    block_off = row_offset // _LN_BLOCK
    out_shape = jax.ShapeDtypeStruct((NUM_TOKENS, HIDDEN), jnp.float32)
    out_spec = pl.BlockSpec(
        (_LN_BLOCK, HIDDEN), lambda i, o=block_off: (o + i, 0))
    x_spec = pl.BlockSpec((_LN_BLOCK, HIDDEN), lambda i: (i, 0))
    w_spec = pl.BlockSpec((1, HIDDEN), lambda i: (0, 0))
    if first:
        return pl.pallas_call(
            _ln_body_first,
            grid=(n_blocks,),
            in_specs=[x_spec, w_spec],
            out_specs=out_spec,
            out_shape=out_shape,
        )(x_group, w2d)
    return pl.pallas_call(
        _ln_body_chain,
        grid=(n_blocks,),
        in_specs=[x_spec, w_spec,
                  pl.BlockSpec(memory_space=pltpu.MemorySpace.HBM)],
        out_specs=out_spec,
        out_shape=out_shape,
        input_output_aliases={2: 0},
    )(x_group, w2d, buf)


def kernel(input_ids, tok_embeddings, norm_weight):
    idx_flat = input_ids.reshape(NUM_TOKENS)
    w2d = norm_weight.reshape(1, HIDDEN)
    offsets = []
    off = 0
    for g in GROUP_SIZES:
        offsets.append(off)
        off += g
    gathered = [
        _sc_gather_group(
            tok_embeddings,
            lax.slice(idx_flat, (offsets[i],), (offsets[i] + GROUP_SIZES[i],)),
            GROUP_SIZES[i])
        for i in range(len(GROUP_SIZES))
    ]
    buf = None
    for i in range(len(GROUP_SIZES)):
        buf = _tc_layernorm_group(i == 0, offsets[i], gathered[i], w2d, buf)
    return buf.reshape(BATCH, SEQ, HIDDEN)


# speedup vs baseline: 1.1509x; 1.0120x over previous
import functools

import jax
import jax.numpy as jnp
from jax import lax
from jax.experimental import pallas as pl
from jax.experimental.pallas import tpu as pltpu
from jax.experimental.pallas import tpu_sc as plsc

VOCAB = 50368
HIDDEN = 768
EPS = 1e-05
BATCH = 4
SEQ = 8192

NUM_TOKENS = BATCH * SEQ
NC = 2
NS = 16
NW = NC * NS
CHUNK = 64

GROUP_SIZES = (8192, 8192, 8192, 8192)
assert sum(GROUP_SIZES) == NUM_TOKENS
assert all(g % (CHUNK * NW) == 0 for g in GROUP_SIZES)


def _sc_gather_group(table, idx_group, group_tokens):
    mesh = plsc.VectorSubcoreMesh(core_axis_name="c", subcore_axis_name="s")
    b_per_w = group_tokens // NW
    n_chunks = b_per_w // CHUNK

    @functools.partial(
        pl.kernel,
        out_type=jax.ShapeDtypeStruct((group_tokens, HIDDEN), jnp.float32),
        mesh=mesh,
        scratch_types=[
            pltpu.VMEM((b_per_w,), jnp.int32),
            pltpu.VMEM((CHUNK, HIDDEN), jnp.float32),
            pltpu.VMEM((CHUNK, HIDDEN), jnp.float32),
            pltpu.SemaphoreType.DMA,
            pltpu.SemaphoreType.DMA,
        ],
    )
    def gather_kernel(table_hbm, idx_hbm, out_hbm, idx_v, rows_a, rows_b, sem_a, sem_b):
        wid = lax.axis_index("s") * NC + lax.axis_index("c")
        base = wid * b_per_w
        pltpu.sync_copy(idx_hbm.at[pl.ds(base, b_per_w)], idx_v)

        bufs = (rows_a, rows_b)
        sems = (sem_a, sem_b)

        def start(c):
            pltpu.async_copy(
                table_hbm.at[idx_v.at[pl.ds(c * CHUNK, CHUNK)]],
                bufs[c % 2], sems[c % 2])

        start(0)
        for c in range(n_chunks):
            pltpu.make_async_copy(
                table_hbm.at[idx_v.at[pl.ds(c * CHUNK, CHUNK)]],
                bufs[c % 2], sems[c % 2]).wait()
            if c + 1 < n_chunks:
                start(c + 1)
            pltpu.sync_copy(bufs[c % 2],
                            out_hbm.at[pl.ds(base + c * CHUNK, CHUNK)])

    return gather_kernel(table, idx_group)


_LN_BLOCK = 1024


def _ln_body_first(x_ref, w_ref, o_ref):
    x = x_ref[...]
    mean = jnp.mean(x, axis=1, keepdims=True)
    xc = x - mean
    var = jnp.mean(xc * xc, axis=1, keepdims=True)
    o_ref[...] = xc * lax.rsqrt(var + EPS) * w_ref[...]


def _ln_body_chain(x_ref, w_ref, buf_ref, o_ref):
    del buf_ref
    _ln_body_first(x_ref, w_ref, o_ref)


def _tc_layernorm_group(first, row_offset, x_group, w2d, buf):
    n_blocks = x_group.shape[0] // _LN_BLOCK
    block_off = row_offset // _LN_BLOCK
    out_shape = jax.ShapeDtypeStruct((NUM_TOKENS, HIDDEN), jnp.float32)
    out_spec = pl.BlockSpec(
        (_LN_BLOCK, HIDDEN), lambda i, o=block_off: (o + i, 0))
    x_spec = pl.BlockSpec((_LN_BLOCK, HIDDEN), lambda i: (i, 0))
    w_spec = pl.BlockSpec((1, HIDDEN), lambda i: (0, 0))
    if first:
        return pl.pallas_call(
            _ln_body_first,
            grid=(n_blocks,),
            in_specs=[x_spec, w_spec],
            out_specs=out_spec,
            out_shape=out_shape,
        )(x_group, w2d)
    return pl.pallas_call(
        _ln_body_chain,
        grid=(n_blocks,),
        in_specs=[x_spec, w_spec,
                  pl.BlockSpec(memory_space=pltpu.MemorySpace.HBM)],
        out_specs=out_spec,
        out_shape=out_shape,
        input_output_aliases={2: 0},
    )(x_group, w2d, buf)


def kernel(input_ids, tok_embeddings, norm_weight):
    idx_flat = input_ids.reshape(NUM_TOKENS)
    w2d = norm_weight.reshape(1, HIDDEN)
    offsets = []
    off = 0
    for g in GROUP_SIZES:
        offsets.append(off)
        off += g
    gathered = [
        _sc_gather_group(
            tok_embeddings,
            lax.slice(idx_flat, (offsets[i],), (offsets[i] + GROUP_SIZES[i],)),
            GROUP_SIZES[i])
        for i in range(len(GROUP_SIZES))
    ]
    buf = None
    for i in range(len(GROUP_SIZES)):
        buf = _tc_layernorm_group(i == 0, offsets[i], gathered[i], w2d, buf)
    return buf.reshape(BATCH, SEQ, HIDDEN)
